# baseline (speedup 1.0000x reference)
import functools

import numpy as np

import jax
import jax.numpy as jnp
from jax import lax
from jax.experimental import pallas as pl
from jax.experimental.pallas import tpu as pltpu
from jax.experimental.pallas import tpu_sc as plsc

_N = 10000
_E = 320000
_C = 128
_K = 128
_NC = 2
_NS = 16
_NW = _NC * _NS
_EPT = _E // _NW
_NPH = 2
_CPP = -(-_EPT // (_NPH * 2 * _K)) * 2
_CPT = _NPH * _CPP
_EPP = _CPP * _K
_EPTP = _CPT * _K
_RPT = 624
_TAIL = _N - _RPT * _NS

_mesh = plsc.VectorSubcoreMesh(core_axis_name="c", subcore_axis_name="s")


@functools.partial(
    pl.kernel,
    out_type=jax.ShapeDtypeStruct((_NC, _N, _C), jnp.float32),
    mesh=_mesh,
    scratch_types=[
        pltpu.VMEM((1, _EPP), jnp.int32),
        pltpu.VMEM((1, _EPP), jnp.int32),
        pltpu.VMEM((_K, _C), jnp.float32),
        pltpu.VMEM((_K, _C), jnp.float32),
        pltpu.VMEM_SHARED((_N + 8, _C), jnp.float32),
        pltpu.SemaphoreType.DMA,
        pltpu.SemaphoreType.DMA,
    ],
)
def _aggregate(h_hbm, src_hbm, dst_hbm, out_hbm, src_v, dst_v, rows_a, rows_b,
               acc, sem_a, sem_b):
    cid = lax.axis_index("c")
    sid = lax.axis_index("s")
    wid = cid * jnp.int32(_NS) + sid
    row0 = sid * jnp.int32(_RPT)

    pltpu.sync_copy(h_hbm.at[pl.ds(row0, _RPT)],
                    acc.at[pl.ds(row0, _RPT)])

    @pl.when(sid == 0)
    def _():
        pltpu.sync_copy(h_hbm.at[pl.ds(_RPT * _NS, _TAIL)],
                        acc.at[pl.ds(_RPT * _NS, _TAIL)])

    plsc.subcore_barrier()

    zero = jnp.int32(0)

    def src_at(c):
        return src_v.at[zero, pl.ds(c * jnp.int32(_K), _K)]

    def dst_at(c):
        return dst_v.at[zero, pl.ds(c * jnp.int32(_K), _K)]

    def body(i, _):
        c0 = i * jnp.int32(2)
        c1 = c0 + jnp.int32(1)
        a = pltpu.async_copy(h_hbm.at[src_at(c0)], rows_a, sem_a)
        b = pltpu.async_copy(h_hbm.at[src_at(c1)], rows_b, sem_b)
        a.wait()
        pltpu.sync_copy(rows_a, acc.at[dst_at(c0)], add=True)
        b.wait()
        pltpu.sync_copy(rows_b, acc.at[dst_at(c1)], add=True)
        return i + jnp.int32(1), None

    for ph in range(_NPH):
        off = jnp.int32(ph * _EPP)
        pltpu.sync_copy(src_hbm.at[wid, pl.ds(zero, 1), pl.ds(off, _EPP)], src_v)
        pltpu.sync_copy(dst_hbm.at[wid, pl.ds(zero, 1), pl.ds(off, _EPP)], dst_v)
        lax.scan(body, jnp.int32(0), None, length=_CPP // 2)

    plsc.subcore_barrier()

    pltpu.sync_copy(acc.at[pl.ds(row0, _RPT)],
                    out_hbm.at[cid, pl.ds(row0, _RPT)])

    @pl.when(sid == 0)
    def _():
        pltpu.sync_copy(acc.at[pl.ds(_RPT * _NS, _TAIL)],
                        out_hbm.at[cid, pl.ds(_RPT * _NS, _TAIL)])


_BLK = 400


def _zero():
    return jnp.int32(0)


def _mm_body(p_ref, h_ref, w_ref, o_ref):
    a = p_ref[0] + p_ref[1] - h_ref[...]
    o_ref[...] = jnp.maximum(
        lax.dot_general(a, w_ref[...], (((1,), (1,)), ((), ())),
                        preferred_element_type=jnp.float32,
                        precision=lax.Precision.HIGHEST),
        0.0)


def _mm(parts, h, w):
    return pl.pallas_call(
        _mm_body,
        grid=(_N // _BLK,),
        in_specs=[
            pl.BlockSpec((_NC, _BLK, _C), lambda i: (_zero(), i, _zero())),
            pl.BlockSpec((_BLK, _C), lambda i: (i, _zero())),
            pl.BlockSpec((_C, _C), lambda i: (_zero(), _zero())),
        ],
        out_specs=pl.BlockSpec((_BLK, _C), lambda i: (i, _zero())),
        out_shape=jax.ShapeDtypeStruct((_N, _C), jnp.float32),
    )(parts, h, w)


def kernel(x, edge_index, W1, W2):
    x = x.astype(jnp.float32)
    pad = ((0, 0), (0, _EPTP - _EPT))
    src = jnp.pad(edge_index[0].astype(jnp.int32).reshape(_NW, _EPT), pad,
                  constant_values=0).reshape(_NW, 1, _EPTP)
    dst = jnp.pad(edge_index[1].astype(jnp.int32).reshape(_NW, _EPT), pad,
                  constant_values=_N).reshape(_NW, 1, _EPTP)
    w1 = W1.astype(jnp.float32)
    w2 = W2.astype(jnp.float32)
    p1 = _aggregate(x, src, dst)
    h1 = _mm(p1, x, w1)
    p2 = _aggregate(h1, src, dst)
    h2 = _mm(p2, h1, w2)
    return h2.astype(jnp.float64)
